# baseline (speedup 1.0000x reference)
import functools

import jax
import jax.numpy as jnp
from jax import lax
from jax.experimental import pallas as pl
from jax.experimental.pallas import tpu as pltpu
from jax.experimental.pallas import tpu_sc as plsc

NTILES = 32
NGRP = 4
CGRP = 8
CH = 256
LANES = 16


def _sc_maxdiff(xsw, idxp, n_tab, n_pad, k_deg):
    words = xsw.shape[1]
    pairs = words // CGRP
    chans = 2 * pairs
    n_sub = n_pad // NGRP
    blocks = n_sub // CH
    k_halves = k_deg // LANES
    mesh = plsc.VectorSubcoreMesh(core_axis_name="c", subcore_axis_name="s")

    @functools.partial(
        pl.kernel,
        out_type=jax.ShapeDtypeStruct((n_pad, 2 * words), jnp.float32),
        mesh=mesh,
        compiler_params=pltpu.CompilerParams(
            needs_layout_passes=False,
            use_tc_tiling_on_sc=False,
        ),
        scratch_types=[
            pltpu.VMEM((n_tab, 8), jnp.int32),
            pltpu.VMEM((CH, k_deg), jnp.int32),
            pltpu.VMEM((CH, k_deg), jnp.int32),
            pltpu.VMEM((CH, k_deg), jnp.int32),
            pltpu.VMEM((CH, k_deg), jnp.int32),
            pltpu.VMEM((CH, chans), jnp.float32),
            pltpu.VMEM((CH, chans), jnp.float32),
            pltpu.SemaphoreType.DMA,
            pltpu.SemaphoreType.DMA,
            pltpu.SemaphoreType.DMA,
            pltpu.SemaphoreType.DMA,
        ],
    )
    def sc_kernel(x_hbm, idx_hbm, m_hbm, table_v,
                  i0a, i1a, i0b, i1b, oba, obb,
                  sem_a, sem_b, sem_oa, sem_ob):
        wid = lax.axis_index("s") * 2 + lax.axis_index("c")
        cg = wid // NGRP
        ng = wid % NGRP
        base_n = ng * n_sub
        pltpu.sync_copy(x_hbm.at[:, pl.ds(cg * pairs, pairs)], table_v)

        def issue_idx(j, d0, d1, sem):
            n0 = base_n + j * CH
            pltpu.async_copy(idx_hbm.at[0, pl.ds(n0, CH), :], d0, sem)
            pltpu.async_copy(idx_hbm.at[1, pl.ds(n0, CH), :], d1, sem)

        def drain_idx(j, d0, d1, sem):
            n0 = base_n + j * CH
            pltpu.make_async_copy(idx_hbm.at[0, pl.ds(n0, CH), :], d0, sem).wait()
            pltpu.make_async_copy(idx_hbm.at[1, pl.ds(n0, CH), :], d1, sem).wait()

        def out_dst(j):
            return m_hbm.at[pl.ds(base_n + j * CH, CH), pl.ds(cg * chans, chans)]

        lane_iota = lax.broadcasted_iota(jnp.int32, (LANES,), 0)

        def compute(j, b0, b1, ob):
            def n_body(v, carry):
                i0h = [b0[v, pl.ds(hh * LANES, LANES)] for hh in range(k_halves)]
                i1h = [b1[v, pl.ds(hh * LANES, LANES)] for hh in range(k_halves)]
                c0h = [(i >> 1) & 7 for i in i0h]
                c1h = [(i >> 1) & 7 for i in i1h]
                row = None
                for p in range(pairs):
                    dm = None
                    for hh in range(k_halves):
                        a = plsc.load_gather(table_v, [i0h[hh], c0h[hh] ^ p])
                        b2 = plsc.load_gather(table_v, [i1h[hh], c1h[hh] ^ p])
                        d = plsc.bitcast(a, jnp.bfloat16) - plsc.bitcast(b2, jnp.bfloat16)
                        dm = d if dm is None else jnp.maximum(dm, d)
                    lo, hi = plsc.unpack(dm, format=plsc.PackFormat.INTERLEAVED)
                    vlo = jnp.broadcast_to(jnp.max(lo), (LANES,))
                    vhi = jnp.broadcast_to(jnp.max(hi), (LANES,))
                    if row is None:
                        row = vlo
                    else:
                        row = jnp.where(lane_iota == 2 * p, vlo, row)
                    row = jnp.where(lane_iota == 2 * p + 1, vhi, row)
                ob[v, :] = row
                return carry

            lax.fori_loop(0, CH, n_body, 0)

        issue_idx(0, i0a, i1a, sem_a)

        def super_body(it, carry):
            ja = 2 * it
            jb = 2 * it + 1
            issue_idx(jb, i0b, i1b, sem_b)
            drain_idx(ja, i0a, i1a, sem_a)

            @pl.when(it > 0)
            def _():
                pltpu.make_async_copy(oba, out_dst(ja - 2), sem_oa).wait()

            compute(ja, i0a, i1a, oba)
            pltpu.async_copy(oba, out_dst(ja), sem_oa)

            @pl.when(it + 1 < blocks // 2)
            def _():
                issue_idx(ja + 2, i0a, i1a, sem_a)

            drain_idx(jb, i0b, i1b, sem_b)

            @pl.when(it > 0)
            def _():
                pltpu.make_async_copy(obb, out_dst(jb - 2), sem_ob).wait()

            compute(jb, i0b, i1b, obb)
            pltpu.async_copy(obb, out_dst(jb), sem_ob)
            return carry

        lax.fori_loop(0, blocks // 2, super_body, 0)
        pltpu.make_async_copy(oba, out_dst(blocks - 2), sem_oa).wait()
        pltpu.make_async_copy(obb, out_dst(blocks - 1), sem_ob).wait()

    return sc_kernel(xsw, idxp)


def _tc_conv(xT, mT, WeT, WoT, br, c, n):
    bn = 1000
    grid = (n // bn,)

    def body(x_ref, m_ref, we_ref, wo_ref, b_ref, y_ref):
        acc = jnp.dot(x_ref[...], we_ref[...], preferred_element_type=jnp.float32)
        acc += jnp.dot(m_ref[...], wo_ref[...], preferred_element_type=jnp.float32)
        y_ref[...] = jnp.maximum(acc + b_ref[...], 0.0)

    return pl.pallas_call(
        body,
        grid=grid,
        in_specs=[
            pl.BlockSpec((bn, c), lambda i: (i, 0)),
            pl.BlockSpec((bn, c), lambda i: (i, 0)),
            pl.BlockSpec((c, c), lambda i: (0, 0)),
            pl.BlockSpec((c, c), lambda i: (0, 0)),
            pl.BlockSpec((1, c), lambda i: (0, 0)),
        ],
        out_specs=pl.BlockSpec((bn, c), lambda i: (i, 0)),
        out_shape=jax.ShapeDtypeStruct((n, c), jnp.float32),
    )(xT, mT, WeT, WoT, br)


def kernel(x, edge_index, W, b):
    B, C, N, _ = x.shape
    K = edge_index.shape[-1]
    n_pad = ((N + (NGRP * CH) - 1) // (NGRP * CH)) * (NGRP * CH)

    xT = jnp.transpose(x[0, :, :, 0])
    xw = lax.bitcast_convert_type(
        xT.astype(jnp.bfloat16).reshape(N, C // 2, 2), jnp.int32
    )
    x8 = xw.reshape(N, C // 16, 8)
    rolls = [
        jnp.stack([x8[:, :, j ^ r] for j in range(8)], axis=-1) for r in range(8)
    ]
    n8 = ((jnp.arange(N) >> 1) & 7)[:, None, None]
    xsw = rolls[7]
    for r in range(6, -1, -1):
        xsw = jnp.where(n8 == r, rolls[r], xsw)
    xsw = xsw.reshape(N, C // 2)

    idxp = jnp.pad(edge_index.reshape(2, N, K), ((0, 0), (0, n_pad - N), (0, 0)))

    mT = _sc_maxdiff(xsw, idxp, N, n_pad, K)

    WT = W.T
    WeT = WT[0::2]
    WoT = WT[1::2]
    yT = _tc_conv(xT, mT, WeT, WoT, b[None, :], C, N)
    return jnp.transpose(yT)[None, :, :, None]


# speedup vs baseline: 1.7784x; 1.7784x over previous
import functools

import jax
import jax.numpy as jnp
from jax import lax
from jax.experimental import pallas as pl
from jax.experimental.pallas import tpu as pltpu
from jax.experimental.pallas import tpu_sc as plsc

NTILES = 32
NGRP = 2
CGRP = 16
CH = 256
LANES = 16


def _sc_maxdiff(xw, idxb, n_tab, n_pad, k_deg):
    pairs = xw.shape[1] // n_tab
    n_sub = n_pad // NGRP
    blocks = n_sub // CH
    groups = CH // LANES
    mesh = plsc.VectorSubcoreMesh(core_axis_name="c", subcore_axis_name="s")

    @functools.partial(
        pl.kernel,
        out_type=jax.ShapeDtypeStruct((2 * CGRP * pairs, n_pad), jnp.float32),
        mesh=mesh,
        compiler_params=pltpu.CompilerParams(
            needs_layout_passes=False,
            use_tc_tiling_on_sc=False,
        ),
        scratch_types=[
            pltpu.VMEM((pairs * n_tab,), jnp.int32),
            pltpu.VMEM((2 * pairs, n_sub), jnp.float32),
            pltpu.VMEM((k_deg, CH), jnp.int32),
            pltpu.VMEM((k_deg, CH), jnp.int32),
            pltpu.VMEM((k_deg, CH), jnp.int32),
            pltpu.VMEM((k_deg, CH), jnp.int32),
            pltpu.SemaphoreType.DMA,
            pltpu.SemaphoreType.DMA,
        ],
    )
    def sc_kernel(x_hbm, idx_hbm, m_hbm, table_v, out_v,
                  i0a, i1a, i0b, i1b, sem_a, sem_b):
        wid = lax.axis_index("s") * 2 + lax.axis_index("c")
        cg = wid // NGRP
        ng = wid % NGRP
        poffs = [jnp.full((LANES,), p * n_tab, jnp.int32) for p in range(pairs)]
        blk0 = ng * blocks

        def issue(blk, d0, d1, sem):
            pltpu.async_copy(idx_hbm.at[0, blk], d0, sem)
            pltpu.async_copy(idx_hbm.at[1, blk], d1, sem)

        def drain(blk, d0, d1, sem):
            pltpu.make_async_copy(idx_hbm.at[0, blk], d0, sem).wait()
            pltpu.make_async_copy(idx_hbm.at[1, blk], d1, sem).wait()

        def compute(j, b0, b1):
            def g_body(g, carry):
                base = g * LANES
                accs = []
                for kk in range(k_deg):
                    i0 = b0[kk, pl.ds(base, LANES)]
                    i1 = b1[kk, pl.ds(base, LANES)]
                    for p in range(pairs):
                        a = plsc.load_gather(table_v, [i0 + poffs[p]])
                        b2 = plsc.load_gather(table_v, [i1 + poffs[p]])
                        d = plsc.bitcast(a, jnp.bfloat16) - plsc.bitcast(b2, jnp.bfloat16)
                        if kk == 0:
                            accs.append(d)
                        else:
                            accs[p] = jnp.maximum(accs[p], d)
                loc = j * CH + base
                for p in range(pairs):
                    lo, hi = plsc.unpack(accs[p], format=plsc.PackFormat.INTERLEAVED)
                    out_v[2 * p, pl.ds(loc, LANES)] = lo
                    out_v[2 * p + 1, pl.ds(loc, LANES)] = hi
                return carry

            lax.fori_loop(0, groups, g_body, 0)

        issue(blk0, i0a, i1a, sem_a)
        pltpu.sync_copy(x_hbm.at[cg], table_v)

        def super_body(it, carry):
            ja = 2 * it
            jb = 2 * it + 1
            issue(blk0 + jb, i0b, i1b, sem_b)
            drain(blk0 + ja, i0a, i1a, sem_a)
            compute(ja, i0a, i1a)

            @pl.when(it + 1 < blocks // 2)
            def _():
                issue(blk0 + ja + 2, i0a, i1a, sem_a)

            drain(blk0 + jb, i0b, i1b, sem_b)
            compute(jb, i0b, i1b)
            return carry

        lax.fori_loop(0, blocks // 2, super_body, 0)
        pltpu.sync_copy(
            out_v,
            m_hbm.at[pl.ds(cg * 2 * pairs, 2 * pairs), pl.ds(ng * n_sub, n_sub)],
        )

    return sc_kernel(xw, idxb)


def _tc_partial(xf, We, b2, c, n):
    def body(x_ref, we_ref, b_ref, y_ref):
        y_ref[...] = (
            jnp.dot(we_ref[...], x_ref[...], preferred_element_type=jnp.float32)
            + b_ref[...]
        )

    return pl.pallas_call(
        body,
        out_shape=jax.ShapeDtypeStruct((c, n), jnp.float32),
    )(xf, We, b2)


def _tc_conv(part, m, Wo, c, n):
    def body(p_ref, m_ref, wo_ref, y_ref):
        acc = p_ref[...] + jnp.dot(
            wo_ref[...], m_ref[:, :n], preferred_element_type=jnp.float32
        )
        y_ref[...] = jnp.maximum(acc, 0.0)

    return pl.pallas_call(
        body,
        out_shape=jax.ShapeDtypeStruct((c, n), jnp.float32),
    )(part, m, Wo)


def kernel(x, edge_index, W, b):
    B, C, N, _ = x.shape
    K = edge_index.shape[-1]
    n_pad = ((N + (NGRP * CH) - 1) // (NGRP * CH)) * (NGRP * CH)
    nblk = n_pad // CH
    pairs = C // (2 * CGRP)

    xf = x.reshape(C, N)
    xb = xf.astype(jnp.bfloat16)
    xwords = lax.bitcast_convert_type(
        xb.reshape(C // 2, 2, N).transpose(0, 2, 1), jnp.int32
    )
    xw = xwords.reshape(CGRP, pairs * N)

    ei = edge_index.reshape(2, N, K)
    eip = jnp.pad(ei, ((0, 0), (0, n_pad - N), (0, 0)))
    idxb = eip.reshape(2, nblk, CH, K).transpose(0, 1, 3, 2)

    m = _sc_maxdiff(xw, idxb, N, n_pad, K)

    We = W[:, 0::2]
    Wo = W[:, 1::2]
    part = _tc_partial(xf, We, b.reshape(C, 1), C, N)
    y = _tc_conv(part, m, Wo, C, N)
    return y.reshape(x.shape)
